# baseline (speedup 1.0000x reference)
import functools

import jax
import jax.numpy as jnp
from jax import lax
from jax.experimental import pallas as pl
from jax.experimental.pallas import tpu as pltpu
from jax.experimental.pallas import tpu_sc as plsc

SENT_N, BATCH_N, L_N = 2048, 512, 8
CH = 128
NBLK = SENT_N // CH
NC, NS = 2, 16
NW = NC * NS
ROWS_PER_W = BATCH_N // NW


LN2 = 0.6931471805599453


def _tc_body(emit_ref, lab_ref, t_ref, part_ref, out_ref,
             alpha_ref, acce_ref, gacc_ref):
    i = pl.program_id(0)
    nblk = pl.num_programs(0)
    rows = lax.broadcasted_iota(jnp.int32, (L_N, BATCH_N), 0)

    Et = jnp.exp(t_ref[...])
    sub = lax.broadcasted_iota(jnp.int32, (L_N, L_N), 0)
    lan = lax.broadcasted_iota(jnp.int32, (L_N, L_N), 1)
    C = [jnp.sum(jnp.where(lan == (sub + d) % L_N, Et, 0.0),
                 axis=1, keepdims=True)
         for d in range(L_N)]

    def substep(t, alpha, gacc):
        emit_t = emit_ref[t]
        lab_t = lab_ref[pl.ds(t, 1), :]
        gacc = gacc + jnp.where(rows == lab_t, emit_t, 0.0)

        w = jnp.exp(emit_t)
        prods = [jnp.roll(alpha, -d, axis=0) * C[d] if d else alpha * C[0]
                 for d in range(L_N)]
        while len(prods) > 1:
            prods = [prods[k] + prods[k + 1] for k in range(0, len(prods), 2)]
        return prods[0] * w, gacc

    def renorm(alpha, acce):
        bits = lax.bitcast_convert_type(alpha[0:1, :], jnp.int32)
        e = lax.shift_right_logical(bits, 23)
        scale = lax.bitcast_convert_type(
            lax.shift_left(254 - e, 23), jnp.float32)
        return alpha * scale, acce + (e - 127)

    KG = 8

    def group(g, carry):
        alpha, acce, gacc = carry
        for dt in range(KG):
            alpha, gacc = substep(g * KG + dt, alpha, gacc)
        alpha, acce = renorm(alpha, acce)
        return alpha, acce, gacc

    zeros_1 = jnp.zeros((1, BATCH_N), jnp.int32)

    @pl.when(i == 0)
    def _first_block():
        emit0 = emit_ref[0]
        lab0 = lab_ref[pl.ds(0, 1), :]
        gacc = jnp.where(rows == lab0, emit0, 0.0)
        alpha = jnp.exp(emit0)
        for dt in range(1, KG):
            alpha, gacc = substep(dt, alpha, gacc)
        alpha, acce = renorm(alpha, zeros_1)
        carry = (alpha, acce, gacc)
        alpha, acce, gacc = lax.fori_loop(1, CH // KG, group, carry)
        alpha_ref[...], acce_ref[...], gacc_ref[...] = alpha, acce, gacc

    @pl.when(i != 0)
    def _rest():
        carry = (alpha_ref[...], acce_ref[...], gacc_ref[...])
        alpha, acce, gacc = lax.fori_loop(0, CH // KG, group, carry)
        alpha_ref[...], acce_ref[...], gacc_ref[...] = alpha, acce, gacc

    @pl.when(i == nblk - 1)
    def _fin():
        alpha = alpha_ref[...]
        lz = (jnp.log(jnp.sum(alpha, axis=0, keepdims=True))
              + LN2 * acce_ref[...].astype(jnp.float32))
        val = (jnp.sum(lz) - jnp.sum(gacc_ref[...])
               - jnp.sum(part_ref[...]))
        out_ref[...] = jnp.full((8, 128), val, jnp.float32)


def _tc_forward(emitT, labT, Tt, partials):
    out = pl.pallas_call(
        _tc_body,
        grid=(NBLK,),
        in_specs=[
            pl.BlockSpec((CH, L_N, BATCH_N), lambda i: (i, 0, 0)),
            pl.BlockSpec((CH, BATCH_N), lambda i: (i, 0)),
            pl.BlockSpec((L_N, L_N), lambda i: (0, 0)),
            pl.BlockSpec((NW, 16), lambda i: (0, 0)),
        ],
        out_specs=pl.BlockSpec((8, 128), lambda i: (0, 0)),
        out_shape=jax.ShapeDtypeStruct((8, 128), jnp.float32),
        scratch_shapes=[
            pltpu.VMEM((L_N, BATCH_N), jnp.float32),
            pltpu.VMEM((1, BATCH_N), jnp.int32),
            pltpu.VMEM((L_N, BATCH_N), jnp.float32),
        ],
    )(emitT, labT, Tt, partials)
    return out[0, 0]


def _sc_gold_trans_build():
    mesh = plsc.VectorSubcoreMesh(core_axis_name="c", subcore_axis_name="s")

    @functools.partial(
        pl.kernel,
        mesh=mesh,
        out_type=jax.ShapeDtypeStruct((NW, 16), jnp.float32),
        scratch_types=[
            pltpu.VMEM((ROWS_PER_W, SENT_N), jnp.int32),
            pltpu.VMEM((64,), jnp.float32),
            pltpu.VMEM((16,), jnp.float32),
        ],
    )
    def sc_kernel(labels_hbm, tflat_hbm, out_hbm, lab_v, t_v, acc_v):
        wid = lax.axis_index("s") * NC + lax.axis_index("c")
        b0 = wid * ROWS_PER_W
        pltpu.sync_copy(labels_hbm.at[pl.ds(b0, ROWS_PER_W)], lab_v)
        pltpu.sync_copy(tflat_hbm, t_v)
        lane = lax.iota(jnp.int32, 16)
        nvec = SENT_N // 16

        dnums = lax.GatherDimensionNumbers(
            offset_dims=(), collapsed_slice_dims=(0,), start_index_map=(0,))

        def _vgather(vec, idx):
            return lax.gather(vec, idx[:, None], dimension_numbers=dnums,
                              slice_sizes=(1,),
                              mode=lax.GatherScatterMode.PROMISE_IN_BOUNDS)

        shift_idx = jnp.minimum(lane + 1, 15)
        zeros_idx = jnp.zeros((16,), jnp.int32)

        def _shift1(a, a_next):
            return jnp.where(lane < 15, _vgather(a, shift_idx),
                             _vgather(a_next, zeros_idx))

        t0 = t_v[pl.ds(0, 16)]
        t1 = t_v[pl.ds(16, 16)]
        t2 = t_v[pl.ds(32, 16)]
        t3 = t_v[pl.ds(48, 16)]

        def _t_lookup(k):
            r = lax.bitwise_and(k, 15)
            q = lax.shift_right_logical(k, 4)
            g01 = jnp.where(q == 0, _vgather(t0, r), _vgather(t1, r))
            g23 = jnp.where(q == 2, _vgather(t2, r), _vgather(t3, r))
            return jnp.where(q < 2, g01, g23)

        def row_loop(j, tot):
            def vec_loop(nv, acc):
                a = lab_v[j, pl.ds(nv * 16, 16)]
                a_next = lab_v[j, pl.ds(nv * 16 + 16, 16)]
                return acc + _t_lookup(a * 8 + _shift1(a, a_next))

            acc = lax.fori_loop(0, nvec - 1, vec_loop, tot)

            a16 = lab_v[j, pl.ds(SENT_N - 16, 16)]
            tv = _t_lookup(a16 * 8 + _shift1(a16, a16))
            return acc + jnp.where(lane < 15, tv, 0.0)

        tot = lax.fori_loop(0, ROWS_PER_W, row_loop,
                            jnp.zeros((16,), jnp.float32))
        acc_v[...] = tot
        pltpu.sync_copy(acc_v, out_hbm.at[wid])

    return sc_kernel


_sc_gold_trans = _sc_gold_trans_build()


def kernel(emit_scores, labels, T):
    emitT = jnp.transpose(emit_scores, (0, 2, 1))
    labT = labels.T
    tflat = T.reshape(-1)
    trans_partials = _sc_gold_trans(labels, tflat)
    return _tc_forward(emitT, labT, T.T, trans_partials)


# speedup vs baseline: 1.0680x; 1.0680x over previous
import functools

import jax
import jax.numpy as jnp
from jax import lax
from jax.experimental import pallas as pl
from jax.experimental.pallas import tpu as pltpu
from jax.experimental.pallas import tpu_sc as plsc

SENT_N, BATCH_N, L_N = 2048, 512, 8
CH = 256
NBLK = SENT_N // CH
NC, NS = 2, 16
NW = NC * NS
ROWS_PER_W = BATCH_N // NW


LN2 = 0.6931471805599453


def _tc_body(emit_ref, lab_ref, t_ref, out_ref,
             alpha_ref, acce_ref, gacc_ref):
    i = pl.program_id(0)
    nblk = pl.num_programs(0)
    rows = lax.broadcasted_iota(jnp.int32, (L_N, BATCH_N), 0)

    Et = jnp.exp(t_ref[...])
    sub = lax.broadcasted_iota(jnp.int32, (L_N, L_N), 0)
    lan = lax.broadcasted_iota(jnp.int32, (L_N, L_N), 1)
    C = [jnp.sum(jnp.where(lan == (sub + d) % L_N, Et, 0.0),
                 axis=1, keepdims=True)
         for d in range(L_N)]

    def substep(t, alpha, gacc):
        emit_t = emit_ref[t]
        lab_t = lab_ref[pl.ds(t, 1), :]
        gacc = gacc + jnp.where(rows == lab_t, emit_t, 0.0)

        w = jnp.exp(emit_t)
        prods = [jnp.roll(alpha, -d, axis=0) * C[d] if d else alpha * C[0]
                 for d in range(L_N)]
        while len(prods) > 1:
            prods = [prods[k] + prods[k + 1] for k in range(0, len(prods), 2)]
        return prods[0] * w, gacc

    def renorm(alpha, acce):
        bits = lax.bitcast_convert_type(alpha[0:1, :], jnp.int32)
        e = lax.shift_right_logical(bits, 23)
        scale = lax.bitcast_convert_type(
            lax.shift_left(254 - e, 23), jnp.float32)
        return alpha * scale, acce + (e - 127)

    KG = 8

    def group(g, carry):
        alpha, acce, gacc = carry
        for dt in range(KG):
            alpha, gacc = substep(g * KG + dt, alpha, gacc)
        alpha, acce = renorm(alpha, acce)
        return alpha, acce, gacc

    zeros_1 = jnp.zeros((1, BATCH_N), jnp.int32)

    @pl.when(i == 0)
    def _first_block():
        emit0 = emit_ref[0]
        lab0 = lab_ref[pl.ds(0, 1), :]
        gacc = jnp.where(rows == lab0, emit0, 0.0)
        alpha = jnp.exp(emit0)
        for dt in range(1, KG):
            alpha, gacc = substep(dt, alpha, gacc)
        alpha, acce = renorm(alpha, zeros_1)
        carry = (alpha, acce, gacc)
        alpha, acce, gacc = lax.fori_loop(1, CH // KG, group, carry)
        alpha_ref[...], acce_ref[...], gacc_ref[...] = alpha, acce, gacc

    @pl.when(i != 0)
    def _rest():
        carry = (alpha_ref[...], acce_ref[...], gacc_ref[...])
        alpha, acce, gacc = lax.fori_loop(0, CH // KG, group, carry)
        alpha_ref[...], acce_ref[...], gacc_ref[...] = alpha, acce, gacc

    @pl.when(i == nblk - 1)
    def _fin():
        alpha = alpha_ref[...]
        lz = (jnp.log(jnp.sum(alpha, axis=0, keepdims=True))
              + LN2 * acce_ref[...].astype(jnp.float32))
        val = jnp.sum(lz) - jnp.sum(gacc_ref[...])
        out_ref[...] = jnp.full((8, 128), val, jnp.float32)


def _tc_forward(emitT, labT, Tt):
    out = pl.pallas_call(
        _tc_body,
        grid=(NBLK,),
        in_specs=[
            pl.BlockSpec((CH, L_N, BATCH_N), lambda i: (i, 0, 0)),
            pl.BlockSpec((CH, BATCH_N), lambda i: (i, 0)),
            pl.BlockSpec((L_N, L_N), lambda i: (0, 0)),
        ],
        out_specs=pl.BlockSpec((8, 128), lambda i: (0, 0)),
        out_shape=jax.ShapeDtypeStruct((8, 128), jnp.float32),
        scratch_shapes=[
            pltpu.VMEM((L_N, BATCH_N), jnp.float32),
            pltpu.VMEM((1, BATCH_N), jnp.int32),
            pltpu.VMEM((L_N, BATCH_N), jnp.float32),
        ],
    )(emitT, labT, Tt)
    return out[0, 0]


def _sc_gold_trans_build():
    mesh = plsc.VectorSubcoreMesh(core_axis_name="c", subcore_axis_name="s")

    @functools.partial(
        pl.kernel,
        mesh=mesh,
        out_type=jax.ShapeDtypeStruct((NW, 16), jnp.float32),
        scratch_types=[
            pltpu.VMEM((ROWS_PER_W, SENT_N), jnp.int32),
            pltpu.VMEM((64,), jnp.float32),
            pltpu.VMEM((16,), jnp.float32),
        ],
    )
    def sc_kernel(labels_hbm, tflat_hbm, out_hbm, lab_v, t_v, acc_v):
        wid = lax.axis_index("s") * NC + lax.axis_index("c")
        b0 = wid * ROWS_PER_W
        pltpu.sync_copy(labels_hbm.at[pl.ds(b0, ROWS_PER_W)], lab_v)
        pltpu.sync_copy(tflat_hbm, t_v)
        lane = lax.iota(jnp.int32, 16)
        nvec = SENT_N // 16

        dnums = lax.GatherDimensionNumbers(
            offset_dims=(), collapsed_slice_dims=(0,), start_index_map=(0,))

        def _vgather(vec, idx):
            return lax.gather(vec, idx[:, None], dimension_numbers=dnums,
                              slice_sizes=(1,),
                              mode=lax.GatherScatterMode.PROMISE_IN_BOUNDS)

        shift_idx = jnp.minimum(lane + 1, 15)
        zeros_idx = jnp.zeros((16,), jnp.int32)

        def _shift1(a, a_next):
            return jnp.where(lane < 15, _vgather(a, shift_idx),
                             _vgather(a_next, zeros_idx))

        t0 = t_v[pl.ds(0, 16)]
        t1 = t_v[pl.ds(16, 16)]
        t2 = t_v[pl.ds(32, 16)]
        t3 = t_v[pl.ds(48, 16)]

        def _t_lookup(k):
            r = lax.bitwise_and(k, 15)
            q = lax.shift_right_logical(k, 4)
            g01 = jnp.where(q == 0, _vgather(t0, r), _vgather(t1, r))
            g23 = jnp.where(q == 2, _vgather(t2, r), _vgather(t3, r))
            return jnp.where(q < 2, g01, g23)

        def row_loop(j, tot):
            def vec_loop(nv, acc):
                a = lab_v[j, pl.ds(nv * 16, 16)]
                a_next = lab_v[j, pl.ds(nv * 16 + 16, 16)]
                return acc + _t_lookup(a * 8 + _shift1(a, a_next))

            acc = lax.fori_loop(0, nvec - 1, vec_loop, tot)

            a16 = lab_v[j, pl.ds(SENT_N - 16, 16)]
            tv = _t_lookup(a16 * 8 + _shift1(a16, a16))
            return acc + jnp.where(lane < 15, tv, 0.0)

        tot = lax.fori_loop(0, ROWS_PER_W, row_loop,
                            jnp.zeros((16,), jnp.float32))
        acc_v[...] = tot
        pltpu.sync_copy(acc_v, out_hbm.at[wid])

    return sc_kernel


_sc_gold_trans = _sc_gold_trans_build()


def kernel(emit_scores, labels, T):
    emitT = jnp.transpose(emit_scores, (0, 2, 1))
    labT = labels.T
    tflat = T.reshape(-1)
    trans_partials = _sc_gold_trans(labels, tflat)
    tc_val = _tc_forward(emitT, labT, T.T)
    return tc_val - jnp.sum(trans_partials)


# speedup vs baseline: 1.0750x; 1.0065x over previous
import functools

import jax
import jax.numpy as jnp
from jax import lax
from jax.experimental import pallas as pl
from jax.experimental.pallas import tpu as pltpu
from jax.experimental.pallas import tpu_sc as plsc

SENT_N, BATCH_N, L_N = 2048, 512, 8
CH = 128
NBLK = SENT_N // CH
NC, NS = 2, 16
NW = NC * NS
ROWS_PER_W = BATCH_N // NW


LN2 = 0.6931471805599453


def _tc_body(emit_ref, lab_ref, t_ref, out_ref,
             alpha_ref, acce_ref, gacc_ref):
    i = pl.program_id(0)
    nblk = pl.num_programs(0)
    rows = lax.broadcasted_iota(jnp.int32, (L_N, BATCH_N), 0)

    Et = jnp.exp(t_ref[...])
    sub = lax.broadcasted_iota(jnp.int32, (L_N, L_N), 0)
    lan = lax.broadcasted_iota(jnp.int32, (L_N, L_N), 1)
    C = [jnp.sum(jnp.where(lan == (sub + d) % L_N, Et, 0.0),
                 axis=1, keepdims=True)
         for d in range(L_N)]

    def substep(emit_t, lab_t, alpha, gacc):
        gacc = gacc + jnp.where(rows == lab_t, emit_t, 0.0)

        w = jnp.exp(emit_t)
        prods = [jnp.roll(alpha, -d, axis=0) * C[d] if d else alpha * C[0]
                 for d in range(L_N)]
        while len(prods) > 1:
            prods = [prods[k] + prods[k + 1] for k in range(0, len(prods), 2)]
        return prods[0] * w, gacc

    def renorm(alpha, acce):
        bits = lax.bitcast_convert_type(alpha[0:1, :], jnp.int32)
        e = lax.shift_right_logical(bits, 23)
        scale = lax.bitcast_convert_type(
            lax.shift_left(254 - e, 23), jnp.float32)
        return alpha * scale, acce + (e - 127)

    KG = 8

    def group(g, carry):
        alpha, acce, gacc = carry
        lab_g = lab_ref[pl.ds(g * KG, KG), :]
        for dt in range(KG):
            emit_t = emit_ref[g * KG + dt]
            alpha, gacc = substep(emit_t, lab_g[dt:dt + 1, :], alpha, gacc)
        alpha, acce = renorm(alpha, acce)
        return alpha, acce, gacc

    zeros_1 = jnp.zeros((1, BATCH_N), jnp.int32)

    @pl.when(i == 0)
    def _first_block():
        emit0 = emit_ref[0]
        lab_g = lab_ref[pl.ds(0, KG), :]
        gacc = jnp.where(rows == lab_g[0:1, :], emit0, 0.0)
        alpha = jnp.exp(emit0)
        for dt in range(1, KG):
            alpha, gacc = substep(emit_ref[dt], lab_g[dt:dt + 1, :],
                                  alpha, gacc)
        alpha, acce = renorm(alpha, zeros_1)
        carry = (alpha, acce, gacc)
        alpha, acce, gacc = lax.fori_loop(1, CH // KG, group, carry)
        alpha_ref[...], acce_ref[...], gacc_ref[...] = alpha, acce, gacc

    @pl.when(i != 0)
    def _rest():
        carry = (alpha_ref[...], acce_ref[...], gacc_ref[...])
        alpha, acce, gacc = lax.fori_loop(0, CH // KG, group, carry)
        alpha_ref[...], acce_ref[...], gacc_ref[...] = alpha, acce, gacc

    @pl.when(i == nblk - 1)
    def _fin():
        alpha = alpha_ref[...]
        lz = (jnp.log(jnp.sum(alpha, axis=0, keepdims=True))
              + LN2 * acce_ref[...].astype(jnp.float32))
        val = jnp.sum(lz) - jnp.sum(gacc_ref[...])
        out_ref[...] = jnp.full((8, 128), val, jnp.float32)


def _tc_forward(emitT, labT, Tt):
    out = pl.pallas_call(
        _tc_body,
        grid=(NBLK,),
        in_specs=[
            pl.BlockSpec((CH, L_N, BATCH_N), lambda i: (i, 0, 0)),
            pl.BlockSpec((CH, BATCH_N), lambda i: (i, 0)),
            pl.BlockSpec((L_N, L_N), lambda i: (0, 0)),
        ],
        out_specs=pl.BlockSpec((8, 128), lambda i: (0, 0)),
        out_shape=jax.ShapeDtypeStruct((8, 128), jnp.float32),
        scratch_shapes=[
            pltpu.VMEM((L_N, BATCH_N), jnp.float32),
            pltpu.VMEM((1, BATCH_N), jnp.int32),
            pltpu.VMEM((L_N, BATCH_N), jnp.float32),
        ],
    )(emitT, labT, Tt)
    return out[0, 0]


def _sc_gold_trans_build():
    mesh = plsc.VectorSubcoreMesh(core_axis_name="c", subcore_axis_name="s")

    @functools.partial(
        pl.kernel,
        mesh=mesh,
        out_type=jax.ShapeDtypeStruct((NW, 16), jnp.float32),
        scratch_types=[
            pltpu.VMEM((ROWS_PER_W, SENT_N), jnp.int32),
            pltpu.VMEM((64,), jnp.float32),
            pltpu.VMEM((16,), jnp.float32),
        ],
    )
    def sc_kernel(labels_hbm, tflat_hbm, out_hbm, lab_v, t_v, acc_v):
        wid = lax.axis_index("s") * NC + lax.axis_index("c")
        b0 = wid * ROWS_PER_W
        pltpu.sync_copy(labels_hbm.at[pl.ds(b0, ROWS_PER_W)], lab_v)
        pltpu.sync_copy(tflat_hbm, t_v)
        lane = lax.iota(jnp.int32, 16)
        nvec = SENT_N // 16

        dnums = lax.GatherDimensionNumbers(
            offset_dims=(), collapsed_slice_dims=(0,), start_index_map=(0,))

        def _vgather(vec, idx):
            return lax.gather(vec, idx[:, None], dimension_numbers=dnums,
                              slice_sizes=(1,),
                              mode=lax.GatherScatterMode.PROMISE_IN_BOUNDS)

        shift_idx = jnp.minimum(lane + 1, 15)
        zeros_idx = jnp.zeros((16,), jnp.int32)

        def _shift1(a, a_next):
            return jnp.where(lane < 15, _vgather(a, shift_idx),
                             _vgather(a_next, zeros_idx))

        t0 = t_v[pl.ds(0, 16)]
        t1 = t_v[pl.ds(16, 16)]
        t2 = t_v[pl.ds(32, 16)]
        t3 = t_v[pl.ds(48, 16)]

        def _t_lookup(k):
            r = lax.bitwise_and(k, 15)
            q = lax.shift_right_logical(k, 4)
            g01 = jnp.where(q == 0, _vgather(t0, r), _vgather(t1, r))
            g23 = jnp.where(q == 2, _vgather(t2, r), _vgather(t3, r))
            return jnp.where(q < 2, g01, g23)

        def row_loop(j, tot):
            def vec_loop(nv, acc):
                a = lab_v[j, pl.ds(nv * 16, 16)]
                a_next = lab_v[j, pl.ds(nv * 16 + 16, 16)]
                return acc + _t_lookup(a * 8 + _shift1(a, a_next))

            acc = lax.fori_loop(0, nvec - 1, vec_loop, tot)

            a16 = lab_v[j, pl.ds(SENT_N - 16, 16)]
            tv = _t_lookup(a16 * 8 + _shift1(a16, a16))
            return acc + jnp.where(lane < 15, tv, 0.0)

        tot = lax.fori_loop(0, ROWS_PER_W, row_loop,
                            jnp.zeros((16,), jnp.float32))
        acc_v[...] = tot
        pltpu.sync_copy(acc_v, out_hbm.at[wid])

    return sc_kernel


_sc_gold_trans = _sc_gold_trans_build()


def kernel(emit_scores, labels, T):
    emitT = jnp.transpose(emit_scores, (0, 2, 1))
    labT = labels.T
    tflat = T.reshape(-1)
    trans_partials = _sc_gold_trans(labels, tflat)
    tc_val = _tc_forward(emitT, labT, T.T)
    return tc_val - jnp.sum(trans_partials)
